# baseline (speedup 1.0000x reference)
import functools

import jax
import jax.numpy as jnp
from jax import lax
from jax.experimental import pallas as pl
from jax.experimental.pallas import tpu as pltpu
from jax.experimental.pallas import tpu_sc as plsc

NUM_TABLES = 26
VOCAB = 100000
DIM = 64
BATCH = 4096
L = 20

NC = 2
NS = 16
NW = NC * NS
LANES = 16
OBW = 2 * DIM

BAGS_PER_W = BATCH // NW
CHUNK = 16
BLOCKS = BAGS_PER_W // CHUNK
ROWS_PER_CHUNK = 2 * CHUNK * L
NGATH = ROWS_PER_CHUNK // 128

GROUPS = ((0, 7), (14, 4), (22, 2))
REPACK_B = 5000


def _make_repack(t0, pairs_g):
  half = pairs_g * VOCAB
  grid = half // REPACK_B
  blk0 = t0 * VOCAB // REPACK_B

  def body(a_ref, b_ref, out_ref):
    out_ref[...] = jnp.concatenate([a_ref[...], b_ref[...]], axis=1)

  return pl.pallas_call(
      body,
      grid=(grid,),
      in_specs=[
          pl.BlockSpec((REPACK_B, DIM), lambda i: (i + blk0, 0)),
          pl.BlockSpec((REPACK_B, DIM), lambda i: (i + blk0 + grid, 0)),
      ],
      out_specs=pl.BlockSpec((REPACK_B, 2 * DIM), lambda i: (i, 0)),
      out_shape=jax.ShapeDtypeStruct((half, 2 * DIM), jnp.float32),
  )


def _make_emb_kernel(t0, pairs_g):
  n_chunks = pairs_g * BLOCKS
  half = pairs_g * VOCAB

  def body(idxn_hbm, tbl_hbm, out_hbm,
           ib0, ib1, gidx0, gidx1, rows0, rows1, ob0, ob1,
           isem0, isem1, gsem0, gsem1, osem0, osem1):
    w = lax.axis_index("s") * NC + lax.axis_index("c")

    def nid_cp(i, ib, sem):
      p = i // BLOCKS
      c = i % BLOCKS
      b0 = w * BAGS_PER_W + c * CHUNK
      return pltpu.make_async_copy(
          idxn_hbm.at[pl.ds(t0 + 2 * p, 2), slice(None), pl.ds(b0, CHUNK)],
          ib, sem)

    def transform(i, ib, gidx):
      p = i // BLOCKS
      consts = []
      for h in range(2):
        lt = 2 * p + h
        consts.append(2 * lt * VOCAB
                      + jnp.where(lt >= pairs_g, 1 - 2 * half, 0))
      for k in range(2 * L):
        h, l = k // L, k % L
        gidx[pl.ds(k * LANES, LANES)] = 2 * ib[h, l, :] + consts[h]

    def gath(gidx, rb, sem, j):
      return pltpu.make_async_copy(
          tbl_hbm.at[gidx.at[pl.ds(j * 128, 128)]],
          rb.at[pl.ds(j * 128, 128)], sem)

    def out_cp(i, ob, sem):
      p = i // BLOCKS
      c = i % BLOCKS
      b0 = w * BAGS_PER_W + c * CHUNK
      return pltpu.make_async_copy(
          ob, out_hbm.at[pl.ds(b0, CHUNK), pl.ds(p * OBW, OBW)], sem)

    def accumulate(rb, ob):
      def bag(c, carry):
        for h in range(2):
          base = h * (CHUNK * L) + c
          for d in range(DIM // LANES):
            acc = rb[base, pl.ds(d * LANES, LANES)]
            for l in range(1, L):
              acc = acc + rb[base + l * CHUNK, pl.ds(d * LANES, LANES)]
            ob[c, pl.ds(h * DIM + d * LANES, LANES)] = acc
        return carry
      lax.fori_loop(0, CHUNK, bag, 0)

    nid_cp(0, ib0, isem0).start()
    nid_cp(0, ib0, isem0).wait()
    transform(0, ib0, gidx0)
    for j in range(NGATH):
      gath(gidx0, rows0, gsem0, j).start()
    nid_cp(1, ib1, isem1).start()

    def step(i2, carry):
      i = i2 * 2

      nid_cp(i + 1, ib1, isem1).wait()
      transform(i + 1, ib1, gidx1)
      for j in range(NGATH):
        gath(gidx1, rows1, gsem1, j).start()
      for j in range(NGATH):
        gath(gidx0, rows0, gsem0, j).wait()

      @pl.when(i + 2 < n_chunks)
      def _():
        nid_cp(i + 2, ib0, isem0).start()

      @pl.when(i >= 2)
      def _():
        out_cp(i - 2, ob0, osem0).wait()

      accumulate(rows0, ob0)
      out_cp(i, ob0, osem0).start()

      @pl.when(i + 2 < n_chunks)
      def _():
        nid_cp(i + 2, ib0, isem0).wait()
        transform(i + 2, ib0, gidx0)
        for j in range(NGATH):
          gath(gidx0, rows0, gsem0, j).start()

      for j in range(NGATH):
        gath(gidx1, rows1, gsem1, j).wait()

      @pl.when(i + 3 < n_chunks)
      def _():
        nid_cp(i + 3, ib1, isem1).start()

      @pl.when(i >= 2)
      def _():
        out_cp(i - 1, ob1, osem1).wait()

      accumulate(rows1, ob1)
      out_cp(i + 1, ob1, osem1).start()
      return carry

    lax.fori_loop(0, n_chunks // 2, step, 0)

    out_cp(n_chunks - 2, ob0, osem0).wait()
    out_cp(n_chunks - 1, ob1, osem1).wait()

  return pl.kernel(
      body,
      out_type=jax.ShapeDtypeStruct((BATCH, pairs_g * OBW), jnp.float32),
      mesh=plsc.VectorSubcoreMesh(
          core_axis_name="c", subcore_axis_name="s",
          num_cores=NC, num_subcores=NS),
      scratch_types=[
          pltpu.VMEM((2, L, CHUNK), jnp.int32),
          pltpu.VMEM((2, L, CHUNK), jnp.int32),
          pltpu.VMEM((ROWS_PER_CHUNK,), jnp.int32),
          pltpu.VMEM((ROWS_PER_CHUNK,), jnp.int32),
          pltpu.VMEM((ROWS_PER_CHUNK, DIM), jnp.float32),
          pltpu.VMEM((ROWS_PER_CHUNK, DIM), jnp.float32),
          pltpu.VMEM((CHUNK, OBW), jnp.float32),
          pltpu.VMEM((CHUNK, OBW), jnp.float32),
          pltpu.SemaphoreType.DMA,
          pltpu.SemaphoreType.DMA,
          pltpu.SemaphoreType.DMA,
          pltpu.SemaphoreType.DMA,
          pltpu.SemaphoreType.DMA,
          pltpu.SemaphoreType.DMA,
      ],
      compiler_params=pltpu.CompilerParams(use_tc_tiling_on_sc=False),
  )


_REPACKS = [_make_repack(t0, pg) for t0, pg in GROUPS]
_EMB_KERNELS = [_make_emb_kernel(t0, pg) for t0, pg in GROUPS]


@jax.jit
def kernel(indices, tables):
  idxn = jnp.transpose(indices.astype(jnp.int32), (0, 2, 1))
  t2d = tables.reshape(NUM_TABLES * VOCAB, DIM)
  outs = []
  for g, (t0, pairs_g) in enumerate(GROUPS):
    tbl = _REPACKS[g](t2d, t2d).reshape(2 * pairs_g * VOCAB, DIM)
    outs.append(_EMB_KERNELS[g](idxn, tbl))
  return jnp.concatenate(outs, axis=1)


# speedup vs baseline: 1.0351x; 1.0351x over previous
import functools

import jax
import jax.numpy as jnp
from jax import lax
from jax.experimental import pallas as pl
from jax.experimental.pallas import tpu as pltpu
from jax.experimental.pallas import tpu_sc as plsc

NUM_TABLES = 26
VOCAB = 100000
DIM = 64
BATCH = 4096
L = 20

NC = 2
NS = 16
NW = NC * NS
LANES = 16
OBW = 2 * DIM

BAGS_PER_W = BATCH // NW
CHUNK = 16
BLOCKS = BAGS_PER_W // CHUNK
ROWS_PER_CHUNK = 2 * CHUNK * L
NGATH = ROWS_PER_CHUNK // 128

GROUPS = ((0, 5), (10, 4), (18, 4))
REPACK_B = 5000


def _make_repack(t0, pairs_g):
  half = pairs_g * VOCAB
  grid = half // REPACK_B
  blk0 = t0 * VOCAB // REPACK_B

  def body(a_ref, b_ref, out_ref):
    out_ref[...] = jnp.concatenate([a_ref[...], b_ref[...]], axis=1)

  return pl.pallas_call(
      body,
      grid=(grid,),
      in_specs=[
          pl.BlockSpec((REPACK_B, DIM), lambda i: (i + blk0, 0)),
          pl.BlockSpec((REPACK_B, DIM), lambda i: (i + blk0 + grid, 0)),
      ],
      out_specs=pl.BlockSpec((REPACK_B, 2 * DIM), lambda i: (i, 0)),
      out_shape=jax.ShapeDtypeStruct((half, 2 * DIM), jnp.float32),
  )


def _make_emb_kernel(t0, pairs_g):
  n_chunks = pairs_g * BLOCKS
  half = pairs_g * VOCAB

  def body(idxn_hbm, tbl_hbm, out_hbm,
           ib0, ib1, gidx0, gidx1, rows0, rows1, ob0, ob1,
           isem0, isem1, gsem0, gsem1, osem0, osem1):
    w = lax.axis_index("s") * NC + lax.axis_index("c")

    def nid_cp(i, ib, sem):
      p = i // BLOCKS
      c = i % BLOCKS
      b0 = w * BAGS_PER_W + c * CHUNK
      return pltpu.make_async_copy(
          idxn_hbm.at[pl.ds(t0 + 2 * p, 2), slice(None), pl.ds(b0, CHUNK)],
          ib, sem)

    def transform(i, ib, gidx):
      p = i // BLOCKS
      consts = []
      for h in range(2):
        lt = 2 * p + h
        consts.append(2 * lt * VOCAB
                      + jnp.where(lt >= pairs_g, 1 - 2 * half, 0))
      for k in range(2 * L):
        h, l = k // L, k % L
        gidx[pl.ds(k * LANES, LANES)] = 2 * ib[h, l, :] + consts[h]

    def gath(gidx, rb, sem, j):
      return pltpu.make_async_copy(
          tbl_hbm.at[gidx.at[pl.ds(j * 128, 128)]],
          rb.at[pl.ds(j * 128, 128)], sem)

    def out_cp(i, ob, sem):
      p = i // BLOCKS
      c = i % BLOCKS
      b0 = w * BAGS_PER_W + c * CHUNK
      return pltpu.make_async_copy(
          ob, out_hbm.at[pl.ds(b0, CHUNK), pl.ds(p * OBW, OBW)], sem)

    def accumulate(rb, ob):
      def bag(c, carry):
        for h in range(2):
          base = h * (CHUNK * L) + c
          for d in range(DIM // LANES):
            acc = rb[base, pl.ds(d * LANES, LANES)]
            for l in range(1, L):
              acc = acc + rb[base + l * CHUNK, pl.ds(d * LANES, LANES)]
            ob[c, pl.ds(h * DIM + d * LANES, LANES)] = acc
        return carry
      lax.fori_loop(0, CHUNK, bag, 0)

    nid_cp(0, ib0, isem0).start()
    nid_cp(0, ib0, isem0).wait()
    transform(0, ib0, gidx0)
    for j in range(NGATH):
      gath(gidx0, rows0, gsem0, j).start()
    nid_cp(1, ib1, isem1).start()

    def step(i2, carry):
      i = i2 * 2

      nid_cp(i + 1, ib1, isem1).wait()
      transform(i + 1, ib1, gidx1)
      for j in range(NGATH):
        gath(gidx1, rows1, gsem1, j).start()
      for j in range(NGATH):
        gath(gidx0, rows0, gsem0, j).wait()

      @pl.when(i + 2 < n_chunks)
      def _():
        nid_cp(i + 2, ib0, isem0).start()

      @pl.when(i >= 2)
      def _():
        out_cp(i - 2, ob0, osem0).wait()

      accumulate(rows0, ob0)
      out_cp(i, ob0, osem0).start()

      @pl.when(i + 2 < n_chunks)
      def _():
        nid_cp(i + 2, ib0, isem0).wait()
        transform(i + 2, ib0, gidx0)
        for j in range(NGATH):
          gath(gidx0, rows0, gsem0, j).start()

      for j in range(NGATH):
        gath(gidx1, rows1, gsem1, j).wait()

      @pl.when(i + 3 < n_chunks)
      def _():
        nid_cp(i + 3, ib1, isem1).start()

      @pl.when(i >= 2)
      def _():
        out_cp(i - 1, ob1, osem1).wait()

      accumulate(rows1, ob1)
      out_cp(i + 1, ob1, osem1).start()
      return carry

    lax.fori_loop(0, n_chunks // 2, step, 0)

    out_cp(n_chunks - 2, ob0, osem0).wait()
    out_cp(n_chunks - 1, ob1, osem1).wait()

  return pl.kernel(
      body,
      out_type=jax.ShapeDtypeStruct((BATCH, pairs_g * OBW), jnp.float32),
      mesh=plsc.VectorSubcoreMesh(
          core_axis_name="c", subcore_axis_name="s",
          num_cores=NC, num_subcores=NS),
      scratch_types=[
          pltpu.VMEM((2, L, CHUNK), jnp.int32),
          pltpu.VMEM((2, L, CHUNK), jnp.int32),
          pltpu.VMEM((ROWS_PER_CHUNK,), jnp.int32),
          pltpu.VMEM((ROWS_PER_CHUNK,), jnp.int32),
          pltpu.VMEM((ROWS_PER_CHUNK, DIM), jnp.float32),
          pltpu.VMEM((ROWS_PER_CHUNK, DIM), jnp.float32),
          pltpu.VMEM((CHUNK, OBW), jnp.float32),
          pltpu.VMEM((CHUNK, OBW), jnp.float32),
          pltpu.SemaphoreType.DMA,
          pltpu.SemaphoreType.DMA,
          pltpu.SemaphoreType.DMA,
          pltpu.SemaphoreType.DMA,
          pltpu.SemaphoreType.DMA,
          pltpu.SemaphoreType.DMA,
      ],
      compiler_params=pltpu.CompilerParams(use_tc_tiling_on_sc=False),
  )


_REPACKS = [_make_repack(t0, pg) for t0, pg in GROUPS]
_EMB_KERNELS = [_make_emb_kernel(t0, pg) for t0, pg in GROUPS]


@jax.jit
def kernel(indices, tables):
  idxn = jnp.transpose(indices.astype(jnp.int32), (0, 2, 1))
  t2d = tables.reshape(NUM_TABLES * VOCAB, DIM)
  outs = []
  for g, (t0, pairs_g) in enumerate(GROUPS):
    tbl = _REPACKS[g](t2d, t2d).reshape(2 * pairs_g * VOCAB, DIM)
    outs.append(_EMB_KERNELS[g](idxn, tbl))
  return jnp.concatenate(outs, axis=1)


# speedup vs baseline: 1.0485x; 1.0130x over previous
import functools

import jax
import jax.numpy as jnp
from jax import lax
from jax.experimental import pallas as pl
from jax.experimental.pallas import tpu as pltpu
from jax.experimental.pallas import tpu_sc as plsc

NUM_TABLES = 26
VOCAB = 100000
DIM = 64
BATCH = 4096
L = 20

NC = 2
NS = 16
NW = NC * NS
LANES = 16
OBW = 2 * DIM

BAGS_PER_W = BATCH // NW
CHUNK = 16
BLOCKS = BAGS_PER_W // CHUNK
ROWS_PER_CHUNK = 2 * CHUNK * L
NGATH = ROWS_PER_CHUNK // 128

GROUPS = ((0, 6), (12, 4), (20, 3))
REPACK_B = 5000


def _make_repack(t0, pairs_g):
  half = pairs_g * VOCAB
  grid = half // REPACK_B
  blk0 = t0 * VOCAB // REPACK_B

  def body(a_ref, b_ref, out_ref):
    out_ref[...] = jnp.concatenate([a_ref[...], b_ref[...]], axis=1)

  return pl.pallas_call(
      body,
      grid=(grid,),
      in_specs=[
          pl.BlockSpec((REPACK_B, DIM), lambda i: (i + blk0, 0)),
          pl.BlockSpec((REPACK_B, DIM), lambda i: (i + blk0 + grid, 0)),
      ],
      out_specs=pl.BlockSpec((REPACK_B, 2 * DIM), lambda i: (i, 0)),
      out_shape=jax.ShapeDtypeStruct((half, 2 * DIM), jnp.float32),
  )


def _make_emb_kernel(t0, pairs_g):
  n_chunks = pairs_g * BLOCKS
  half = pairs_g * VOCAB

  def body(idxn_hbm, tbl_hbm, out_hbm,
           ib0, ib1, gidx0, gidx1, rows0, rows1, ob0, ob1,
           isem0, isem1, gsem0, gsem1, osem0, osem1):
    w = lax.axis_index("s") * NC + lax.axis_index("c")

    def nid_cp(i, ib, sem):
      p = i // BLOCKS
      c = i % BLOCKS
      b0 = w * BAGS_PER_W + c * CHUNK
      return pltpu.make_async_copy(
          idxn_hbm.at[pl.ds(t0 + 2 * p, 2), slice(None), pl.ds(b0, CHUNK)],
          ib, sem)

    def transform(i, ib, gidx):
      p = i // BLOCKS
      consts = []
      for h in range(2):
        lt = 2 * p + h
        consts.append(2 * lt * VOCAB
                      + jnp.where(lt >= pairs_g, 1 - 2 * half, 0))
      for k in range(2 * L):
        h, l = k // L, k % L
        gidx[pl.ds(k * LANES, LANES)] = 2 * ib[h, l, :] + consts[h]

    def gath(gidx, rb, sem, j):
      return pltpu.make_async_copy(
          tbl_hbm.at[gidx.at[pl.ds(j * 128, 128)]],
          rb.at[pl.ds(j * 128, 128)], sem)

    def out_cp(i, ob, sem):
      p = i // BLOCKS
      c = i % BLOCKS
      b0 = w * BAGS_PER_W + c * CHUNK
      return pltpu.make_async_copy(
          ob, out_hbm.at[pl.ds(b0, CHUNK), pl.ds(p * OBW, OBW)], sem)

    def accumulate(rb, ob):
      def bag(c, carry):
        for h in range(2):
          base = h * (CHUNK * L) + c
          for d in range(DIM // LANES):
            acc = rb[base, pl.ds(d * LANES, LANES)]
            for l in range(1, L):
              acc = acc + rb[base + l * CHUNK, pl.ds(d * LANES, LANES)]
            ob[c, pl.ds(h * DIM + d * LANES, LANES)] = acc
        return carry
      lax.fori_loop(0, CHUNK, bag, 0)

    nid_cp(0, ib0, isem0).start()
    nid_cp(0, ib0, isem0).wait()
    transform(0, ib0, gidx0)
    for j in range(NGATH):
      gath(gidx0, rows0, gsem0, j).start()
    nid_cp(1, ib1, isem1).start()

    def step(i2, carry):
      i = i2 * 2

      nid_cp(i + 1, ib1, isem1).wait()
      transform(i + 1, ib1, gidx1)
      for j in range(NGATH):
        gath(gidx1, rows1, gsem1, j).start()
      for j in range(NGATH):
        gath(gidx0, rows0, gsem0, j).wait()

      @pl.when(i + 2 < n_chunks)
      def _():
        nid_cp(i + 2, ib0, isem0).start()

      @pl.when(i >= 2)
      def _():
        out_cp(i - 2, ob0, osem0).wait()

      accumulate(rows0, ob0)
      out_cp(i, ob0, osem0).start()

      @pl.when(i + 2 < n_chunks)
      def _():
        nid_cp(i + 2, ib0, isem0).wait()
        transform(i + 2, ib0, gidx0)
        for j in range(NGATH):
          gath(gidx0, rows0, gsem0, j).start()

      for j in range(NGATH):
        gath(gidx1, rows1, gsem1, j).wait()

      @pl.when(i + 3 < n_chunks)
      def _():
        nid_cp(i + 3, ib1, isem1).start()

      @pl.when(i >= 2)
      def _():
        out_cp(i - 1, ob1, osem1).wait()

      accumulate(rows1, ob1)
      out_cp(i + 1, ob1, osem1).start()
      return carry

    lax.fori_loop(0, n_chunks // 2, step, 0)

    out_cp(n_chunks - 2, ob0, osem0).wait()
    out_cp(n_chunks - 1, ob1, osem1).wait()

  return pl.kernel(
      body,
      out_type=jax.ShapeDtypeStruct((BATCH, pairs_g * OBW), jnp.float32),
      mesh=plsc.VectorSubcoreMesh(
          core_axis_name="c", subcore_axis_name="s",
          num_cores=NC, num_subcores=NS),
      scratch_types=[
          pltpu.VMEM((2, L, CHUNK), jnp.int32),
          pltpu.VMEM((2, L, CHUNK), jnp.int32),
          pltpu.VMEM((ROWS_PER_CHUNK,), jnp.int32),
          pltpu.VMEM((ROWS_PER_CHUNK,), jnp.int32),
          pltpu.VMEM((ROWS_PER_CHUNK, DIM), jnp.float32),
          pltpu.VMEM((ROWS_PER_CHUNK, DIM), jnp.float32),
          pltpu.VMEM((CHUNK, OBW), jnp.float32),
          pltpu.VMEM((CHUNK, OBW), jnp.float32),
          pltpu.SemaphoreType.DMA,
          pltpu.SemaphoreType.DMA,
          pltpu.SemaphoreType.DMA,
          pltpu.SemaphoreType.DMA,
          pltpu.SemaphoreType.DMA,
          pltpu.SemaphoreType.DMA,
      ],
      compiler_params=pltpu.CompilerParams(use_tc_tiling_on_sc=False),
  )


_REPACKS = [_make_repack(t0, pg) for t0, pg in GROUPS]
_EMB_KERNELS = [_make_emb_kernel(t0, pg) for t0, pg in GROUPS]


@jax.jit
def kernel(indices, tables):
  idxn = jnp.transpose(indices.astype(jnp.int32), (0, 2, 1))
  t2d = tables.reshape(NUM_TABLES * VOCAB, DIM)
  outs = []
  for g, (t0, pairs_g) in enumerate(GROUPS):
    tbl = _REPACKS[g](t2d, t2d).reshape(2 * pairs_g * VOCAB, DIM)
    outs.append(_EMB_KERNELS[g](idxn, tbl))
  return jnp.concatenate(outs, axis=1)


# speedup vs baseline: 1.0555x; 1.0067x over previous
import functools

import jax
import jax.numpy as jnp
from jax import lax
from jax.experimental import pallas as pl
from jax.experimental.pallas import tpu as pltpu
from jax.experimental.pallas import tpu_sc as plsc

NUM_TABLES = 26
VOCAB = 100000
DIM = 64
BATCH = 4096
L = 20

NC = 2
NS = 16
NW = NC * NS
LANES = 16
OBW = 2 * DIM

BAGS_PER_W = BATCH // NW
CHUNK = 16
BLOCKS = BAGS_PER_W // CHUNK
ROWS_PER_CHUNK = 2 * CHUNK * L
NGATH = ROWS_PER_CHUNK // 128

GROUPS = ((0, 6), (12, 4), (20, 3))
REPACK_B = 10000


def _make_repack(t0, pairs_g):
  half = pairs_g * VOCAB
  grid = half // REPACK_B
  blk0 = t0 * VOCAB // REPACK_B

  def body(a_ref, b_ref, out_ref):
    out_ref[...] = jnp.concatenate([a_ref[...], b_ref[...]], axis=1)

  return pl.pallas_call(
      body,
      grid=(grid,),
      in_specs=[
          pl.BlockSpec((REPACK_B, DIM), lambda i: (i + blk0, 0)),
          pl.BlockSpec((REPACK_B, DIM), lambda i: (i + blk0 + grid, 0)),
      ],
      out_specs=pl.BlockSpec((REPACK_B, 2 * DIM), lambda i: (i, 0)),
      out_shape=jax.ShapeDtypeStruct((half, 2 * DIM), jnp.float32),
  )


def _make_emb_kernel(t0, pairs_g):
  n_chunks = pairs_g * BLOCKS
  half = pairs_g * VOCAB

  def body(idxn_hbm, tbl_hbm, out_hbm,
           ib0, ib1, gidx0, gidx1, rows0, rows1, ob0, ob1,
           isem0, isem1, gsem0, gsem1, osem0, osem1):
    w = lax.axis_index("s") * NC + lax.axis_index("c")

    def nid_cp(i, ib, sem):
      p = i // BLOCKS
      c = i % BLOCKS
      b0 = w * BAGS_PER_W + c * CHUNK
      return pltpu.make_async_copy(
          idxn_hbm.at[pl.ds(t0 + 2 * p, 2), slice(None), pl.ds(b0, CHUNK)],
          ib, sem)

    def transform(i, ib, gidx):
      p = i // BLOCKS
      consts = []
      for h in range(2):
        lt = 2 * p + h
        consts.append(2 * lt * VOCAB
                      + jnp.where(lt >= pairs_g, 1 - 2 * half, 0))
      for k in range(2 * L):
        h, l = k // L, k % L
        gidx[pl.ds(k * LANES, LANES)] = 2 * ib[h, l, :] + consts[h]

    def gath(gidx, rb, sem, j):
      return pltpu.make_async_copy(
          tbl_hbm.at[gidx.at[pl.ds(j * 128, 128)]],
          rb.at[pl.ds(j * 128, 128)], sem)

    def out_cp(i, ob, sem):
      p = i // BLOCKS
      c = i % BLOCKS
      b0 = w * BAGS_PER_W + c * CHUNK
      return pltpu.make_async_copy(
          ob, out_hbm.at[pl.ds(b0, CHUNK), pl.ds(p * OBW, OBW)], sem)

    def accumulate(rb, ob):
      def bag(c, carry):
        for h in range(2):
          base = h * (CHUNK * L) + c
          for d in range(DIM // LANES):
            acc = rb[base, pl.ds(d * LANES, LANES)]
            for l in range(1, L):
              acc = acc + rb[base + l * CHUNK, pl.ds(d * LANES, LANES)]
            ob[c, pl.ds(h * DIM + d * LANES, LANES)] = acc
        return carry
      lax.fori_loop(0, CHUNK, bag, 0)

    nid_cp(0, ib0, isem0).start()
    nid_cp(0, ib0, isem0).wait()
    transform(0, ib0, gidx0)
    for j in range(NGATH):
      gath(gidx0, rows0, gsem0, j).start()
    nid_cp(1, ib1, isem1).start()

    def step(i2, carry):
      i = i2 * 2

      nid_cp(i + 1, ib1, isem1).wait()
      transform(i + 1, ib1, gidx1)
      for j in range(NGATH):
        gath(gidx1, rows1, gsem1, j).start()
      for j in range(NGATH):
        gath(gidx0, rows0, gsem0, j).wait()

      @pl.when(i + 2 < n_chunks)
      def _():
        nid_cp(i + 2, ib0, isem0).start()

      @pl.when(i >= 2)
      def _():
        out_cp(i - 2, ob0, osem0).wait()

      accumulate(rows0, ob0)
      out_cp(i, ob0, osem0).start()

      @pl.when(i + 2 < n_chunks)
      def _():
        nid_cp(i + 2, ib0, isem0).wait()
        transform(i + 2, ib0, gidx0)
        for j in range(NGATH):
          gath(gidx0, rows0, gsem0, j).start()

      for j in range(NGATH):
        gath(gidx1, rows1, gsem1, j).wait()

      @pl.when(i + 3 < n_chunks)
      def _():
        nid_cp(i + 3, ib1, isem1).start()

      @pl.when(i >= 2)
      def _():
        out_cp(i - 1, ob1, osem1).wait()

      accumulate(rows1, ob1)
      out_cp(i + 1, ob1, osem1).start()
      return carry

    lax.fori_loop(0, n_chunks // 2, step, 0)

    out_cp(n_chunks - 2, ob0, osem0).wait()
    out_cp(n_chunks - 1, ob1, osem1).wait()

  return pl.kernel(
      body,
      out_type=jax.ShapeDtypeStruct((BATCH, pairs_g * OBW), jnp.float32),
      mesh=plsc.VectorSubcoreMesh(
          core_axis_name="c", subcore_axis_name="s",
          num_cores=NC, num_subcores=NS),
      scratch_types=[
          pltpu.VMEM((2, L, CHUNK), jnp.int32),
          pltpu.VMEM((2, L, CHUNK), jnp.int32),
          pltpu.VMEM((ROWS_PER_CHUNK,), jnp.int32),
          pltpu.VMEM((ROWS_PER_CHUNK,), jnp.int32),
          pltpu.VMEM((ROWS_PER_CHUNK, DIM), jnp.float32),
          pltpu.VMEM((ROWS_PER_CHUNK, DIM), jnp.float32),
          pltpu.VMEM((CHUNK, OBW), jnp.float32),
          pltpu.VMEM((CHUNK, OBW), jnp.float32),
          pltpu.SemaphoreType.DMA,
          pltpu.SemaphoreType.DMA,
          pltpu.SemaphoreType.DMA,
          pltpu.SemaphoreType.DMA,
          pltpu.SemaphoreType.DMA,
          pltpu.SemaphoreType.DMA,
      ],
      compiler_params=pltpu.CompilerParams(use_tc_tiling_on_sc=False),
  )


_REPACKS = [_make_repack(t0, pg) for t0, pg in GROUPS]
_EMB_KERNELS = [_make_emb_kernel(t0, pg) for t0, pg in GROUPS]


@jax.jit
def kernel(indices, tables):
  idxn = jnp.transpose(indices.astype(jnp.int32), (0, 2, 1))
  t2d = tables.reshape(NUM_TABLES * VOCAB, DIM)
  outs = []
  for g, (t0, pairs_g) in enumerate(GROUPS):
    tbl = _REPACKS[g](t2d, t2d).reshape(2 * pairs_g * VOCAB, DIM)
    outs.append(_EMB_KERNELS[g](idxn, tbl))
  return jnp.concatenate(outs, axis=1)
